# baseline (speedup 1.0000x reference)
import functools

import jax
import jax.numpy as jnp
from jax import lax
from jax.experimental import pallas as pl
from jax.experimental.pallas import tpu as pltpu
from jax.experimental.pallas import tpu_sc as plsc

F = 4
B = 4096
L = 12
V = 100000
D = 64
LANES = 16

NW = 32
ROWS_PER_W = B // NW
RC = 16
NCHUNK = ROWS_PER_W // RC
CB = RC * F
IPC = CB * L
SEG = RC * L
JW = 64
NJ = SEG // JW


def _sc_body(tab, pw, idx, lens, pred, idx_v, rows, len_v, pw_v, out_v, sem):
    wid = lax.axis_index("c") * 16 + lax.axis_index("s")
    row_base = wid * ROWS_PER_W
    pltpu.sync_copy(pw, pw_v)

    def chunk_body(ci, carry):
        row0 = row_base + ci * RC
        for f in range(F):
            pltpu.sync_copy(lens.at[pl.ds(f * B + row0, RC)],
                            len_v.at[pl.ds(f * RC, RC)])
            for j in range(NJ):
                pltpu.sync_copy(
                    idx.at[pl.ds((f * B + row0) * L + j * JW, JW)],
                    idx_v.at[f * NJ + j])
        copies = [
            pltpu.async_copy(tab.at[f].at[idx_v.at[f * NJ + j]],
                             rows.at[pl.ds((f * NJ + j) * JW, JW)], sem)
            for f in range(F) for j in range(NJ)
        ]
        for cpy in copies:
            cpy.wait()

        def feat_body(g, carry2):
            pwg = pw_v[pl.ds(g * LANES, LANES)]
            pw_s = [pwg[l] for l in range(L)]
            len16 = len_v[pl.ds(g * RC, RC)]
            for b2 in range(RC):
                ln = len16[b2]
                base = (g * RC + b2) * L
                accs = [None] * (D // LANES)
                for l in range(L):
                    w_l = jnp.where(l < ln, pw_s[l], 0.0)
                    for c in range(D // LANES):
                        t = w_l * rows[base + l, pl.ds(c * LANES, LANES)]
                        accs[c] = t if accs[c] is None else accs[c] + t
                for c in range(D // LANES):
                    out_v[b2, pl.ds(g * D + c * LANES, LANES)] = accs[c]
            return carry2

        lax.fori_loop(0, F, feat_body, 0)
        pltpu.sync_copy(out_v, pred.at[pl.ds(row0, RC)])
        return carry

    lax.fori_loop(0, NCHUNK, chunk_body, 0)


def _sc_pooled(tables, pw_pad, idx_flat, lens_flat):
    mesh = plsc.VectorSubcoreMesh(core_axis_name="c", subcore_axis_name="s")
    run = functools.partial(
        pl.kernel,
        mesh=mesh,
        compiler_params=pltpu.CompilerParams(use_tc_tiling_on_sc=False),
        out_type=jax.ShapeDtypeStruct((B, F * D), jnp.float32),
        scratch_types=[
            pltpu.VMEM((F * NJ, JW), jnp.int32),
            pltpu.VMEM((IPC, D), jnp.float32),
            pltpu.VMEM((CB,), jnp.int32),
            pltpu.VMEM((F * LANES,), jnp.float32),
            pltpu.VMEM((RC, F * D), jnp.float32),
            pltpu.SemaphoreType.DMA,
        ],
    )(_sc_body)
    return run(tables, pw_pad, idx_flat, lens_flat)


def kernel(tables, pos_weight, indices, lengths):
    pw_pad = jnp.zeros((F, LANES), jnp.float32).at[:, :L].set(
        pos_weight.astype(jnp.float32)).reshape(F * LANES)
    idx_flat = indices.astype(jnp.int32).reshape(F * B * L)
    lens_flat = lengths.astype(jnp.int32).reshape(F * B)
    pred = _sc_pooled(tables, pw_pad, idx_flat, lens_flat)
    loss = jnp.mean(pred)
    return (loss, pred)


# speedup vs baseline: 1.1154x; 1.1154x over previous
import functools

import jax
import jax.numpy as jnp
from jax import lax
from jax.experimental import pallas as pl
from jax.experimental.pallas import tpu as pltpu
from jax.experimental.pallas import tpu_sc as plsc

F = 4
B = 4096
L = 12
V = 100000
D = 64
LANES = 16

NW = 32
ROWS_PER_W = B // NW
RC = 16
NCHUNK = ROWS_PER_W // RC
CB = RC * F
IPC = CB * L
SEG = RC * L
NJ = IPC // 128


def _sc_body(tab, pw, idx, lens, pred, idx_raw, idx_adj, rows, len_v, pw_v,
             out_v, sem_a, sem_b):
    wid = lax.axis_index("c") * 16 + lax.axis_index("s")
    row_base = wid * ROWS_PER_W
    sems = (sem_a, sem_b)
    pltpu.sync_copy(pw, pw_v)

    def stage(ci, buf):
        row0 = row_base + ci * RC
        for f in range(F):
            pltpu.sync_copy(lens.at[pl.ds(f * B + row0, RC)],
                            len_v.at[buf].at[pl.ds(f * RC, RC)])
            pltpu.sync_copy(idx.at[pl.ds((f * B + row0) * L, SEG)],
                            idx_raw.at[pl.ds(f * SEG, SEG)])
        for k in range(IPC // LANES):
            idx_adj[buf, k // 8, pl.ds((k % 8) * LANES, LANES)] = (
                idx_raw[pl.ds(k * LANES, LANES)]
                + (k // (SEG // LANES)) * V)
        return [
            pltpu.async_copy(tab.at[idx_adj.at[buf, j]],
                             rows.at[buf].at[pl.ds(j * 128, 128)],
                             sems[buf])
            for j in range(NJ)
        ]

    def drain(buf):
        for j in range(NJ):
            pltpu.make_async_copy(tab.at[idx_adj.at[buf, j]],
                                  rows.at[buf].at[pl.ds(j * 128, 128)],
                                  sems[buf]).wait()

    def compute(ci, buf):
        row0 = row_base + ci * RC

        def feat_body(g, carry2):
            pwg = pw_v[pl.ds(g * LANES, LANES)]
            pw_s = [pwg[l] for l in range(L)]
            len16 = len_v[buf, pl.ds(g * RC, RC)]
            for b2 in range(RC):
                ln = len16[b2]
                base = (g * RC + b2) * L
                accs = [None] * (D // LANES)
                for l in range(L):
                    w_l = jnp.where(l < ln, pw_s[l], 0.0)
                    for c in range(D // LANES):
                        t = w_l * rows[buf, base + l, pl.ds(c * LANES, LANES)]
                        accs[c] = t if accs[c] is None else accs[c] + t
                for c in range(D // LANES):
                    out_v[b2, pl.ds(g * D + c * LANES, LANES)] = accs[c]
            return carry2

        lax.fori_loop(0, F, feat_body, 0)
        pltpu.sync_copy(out_v, pred.at[pl.ds(row0, RC)])

    stage(0, 0)
    def pair_body(i, carry):
        ci = 2 * i
        stage(ci + 1, 1)
        drain(0)
        compute(ci, 0)
        stage(ci + 2, 0)
        drain(1)
        compute(ci + 1, 1)
        return carry
    lax.fori_loop(0, NCHUNK // 2 - 1, pair_body, 0)
    stage(NCHUNK - 1, 1)
    drain(0)
    compute(NCHUNK - 2, 0)
    drain(1)
    compute(NCHUNK - 1, 1)


def _sc_pooled(tables_flat, pw_pad, idx_flat, lens_flat):
    mesh = plsc.VectorSubcoreMesh(core_axis_name="c", subcore_axis_name="s")
    run = functools.partial(
        pl.kernel,
        mesh=mesh,
        compiler_params=pltpu.CompilerParams(use_tc_tiling_on_sc=False),
        out_type=jax.ShapeDtypeStruct((B, F * D), jnp.float32),
        scratch_types=[
            pltpu.VMEM((IPC,), jnp.int32),
            pltpu.VMEM((2, NJ, 128), jnp.int32),
            pltpu.VMEM((2, IPC, D), jnp.float32),
            pltpu.VMEM((2, CB), jnp.int32),
            pltpu.VMEM((F * LANES,), jnp.float32),
            pltpu.VMEM((RC, F * D), jnp.float32),
            pltpu.SemaphoreType.DMA,
            pltpu.SemaphoreType.DMA,
        ],
    )(_sc_body)
    return run(tables_flat, pw_pad, idx_flat, lens_flat)


def kernel(tables, pos_weight, indices, lengths):
    tables_flat = tables.reshape(F * V, D)
    pw_pad = jnp.zeros((F, LANES), jnp.float32).at[:, :L].set(
        pos_weight.astype(jnp.float32)).reshape(F * LANES)
    idx_flat = indices.astype(jnp.int32).reshape(F * B * L)
    lens_flat = lengths.astype(jnp.int32).reshape(F * B)
    pred = _sc_pooled(tables_flat, pw_pad, idx_flat, lens_flat)
    loss = jnp.mean(pred)
    return (loss, pred)
